# baseline (speedup 1.0000x reference)
import functools

import jax
import jax.numpy as jnp
from jax import lax
from jax.experimental import pallas as pl
from jax.experimental.pallas import tpu as pltpu
from jax.experimental.pallas import tpu_sc as plsc

IN_DIM = 16384
OUT_DIM = 16384
BATCH = 1024

NUM_CORES = 2
NUM_SUBCORES = 16
NUM_WORKERS = NUM_CORES * NUM_SUBCORES
ROWS_PER_WORKER = BATCH // NUM_WORKERS
PAIRS_PER_WORKER = ROWS_PER_WORKER // 2
LANES = 16
GROUPS = OUT_DIM // LANES
CHUNK_W = 2048
CHUNK_GROUPS = CHUNK_W // LANES
NCHUNKS = OUT_DIM // CHUNK_W
UNROLL = 6


def _coef_body(wt_ref, a_ref, b_ref, c01_ref, c23_ref, ab_ref):
    w = wt_ref[...]
    m = jnp.max(w, axis=0, keepdims=True)
    e = jnp.exp(w - m)
    p = e / jnp.sum(e, axis=0, keepdims=True)
    r = [p[i:i + 1, :] for i in range(16)]
    c0 = r[8] + r[9] + r[10] + r[11] + r[12] + r[13] + r[14] + r[15]
    c1 = (r[2] + r[3] + r[6] + r[7]) - (r[8] + r[9] + r[12] + r[13])
    c2 = (r[4] + r[5] + r[6] + r[7]) - (r[8] + r[9] + r[10] + r[11])
    c3 = (r[1] - r[2] - r[4] - 2.0 * r[6] - r[7]
          + r[8] + 2.0 * r[9] + r[11] + r[13] - r[14])

    def pack_pair(lo, hi):
        lo_b = lax.bitcast_convert_type(lo.astype(jnp.bfloat16), jnp.uint16)
        hi_b = lax.bitcast_convert_type(hi.astype(jnp.bfloat16), jnp.uint16)
        word = lax.bitwise_or(lo_b.astype(jnp.uint32),
                              lax.shift_left(hi_b.astype(jnp.uint32),
                                             jnp.uint32(16)))
        return lax.bitcast_convert_type(word, jnp.int32)

    c01_ref[...] = pack_pair(c0, c1)
    c23_ref[...] = pack_pair(c2, c3)
    ab_ref[...] = lax.bitwise_or(a_ref[...], lax.shift_left(b_ref[...], 14))


def _coefficients(weights, a, b):
    wt = weights.T
    c01, c23, ab = pl.pallas_call(
        _coef_body,
        out_shape=[
            jax.ShapeDtypeStruct((1, OUT_DIM), jnp.int32),
            jax.ShapeDtypeStruct((1, OUT_DIM), jnp.int32),
            jax.ShapeDtypeStruct((1, OUT_DIM), jnp.int32),
        ],
    )(wt, a.reshape(1, OUT_DIM), b.reshape(1, OUT_DIM))
    return c01.reshape(OUT_DIM), c23.reshape(OUT_DIM), ab.reshape(OUT_DIM)


_HI_MASK = -65536


@functools.partial(
    pl.kernel,
    mesh=plsc.VectorSubcoreMesh(core_axis_name="c", subcore_axis_name="s"),
    out_type=jax.ShapeDtypeStruct((BATCH, OUT_DIM), jnp.float32),
    compiler_params=pltpu.CompilerParams(needs_layout_passes=False),
    scratch_types=[
        pltpu.VMEM((OUT_DIM,), jnp.int32),
        pltpu.VMEM((OUT_DIM,), jnp.int32),
        pltpu.VMEM((OUT_DIM,), jnp.int32),
        pltpu.VMEM((IN_DIM,), jnp.float32),
        pltpu.VMEM((IN_DIM,), jnp.float32),
        pltpu.VMEM((IN_DIM,), jnp.float32),
        pltpu.VMEM((IN_DIM,), jnp.float32),
        pltpu.VMEM((CHUNK_W,), jnp.float32),
        pltpu.VMEM((CHUNK_W,), jnp.float32),
        pltpu.VMEM((CHUNK_W,), jnp.float32),
        pltpu.VMEM((CHUNK_W,), jnp.float32),
        pltpu.SemaphoreType.DMA,
        pltpu.SemaphoreType.DMA,
        pltpu.SemaphoreType.DMA,
        pltpu.SemaphoreType.DMA,
    ],
)
def _sc_gather_combine(x_hbm, ab_hbm, c01_hbm, c23_hbm, out_hbm,
                       ab_v, c01_v, c23_v,
                       x00, x01, x10, x11,
                       o00, o01, o10, o11,
                       in_sem0, in_sem1, out_sem0, out_sem1):
    wid = lax.axis_index("s") * NUM_CORES + lax.axis_index("c")
    base = wid * ROWS_PER_WORKER

    pltpu.sync_copy(ab_hbm, ab_v)
    pltpu.sync_copy(c01_hbm, c01_v)
    pltpu.sync_copy(c23_hbm, c23_v)

    xbufs = ((x00, x01), (x10, x11))
    obufs = ((o00, o10), (o01, o11))
    in_sems = (in_sem0, in_sem1)
    out_sems = (out_sem0, out_sem1)

    def in_copy(pair, buf):
        r0 = base + pair * 2
        return (
            pltpu.make_async_copy(x_hbm.at[r0], xbufs[buf][0], in_sems[buf]),
            pltpu.make_async_copy(x_hbm.at[r0 + 1], xbufs[buf][1], in_sems[buf]),
        )

    def start_in(pair, buf):
        for c in in_copy(pair, buf):
            c.start()

    def wait_in(pair, buf):
        for c in in_copy(pair, buf):
            c.wait()

    start_in(0, 0)

    def compute_chunk(xb0, xb1, slot, ch):
        ob0, ob1 = obufs[slot]

        @plsc.parallel_loop(0, CHUNK_GROUPS, unroll=UNROLL)
        def grp_fn(i):
            off = i * LANES
            g = ch * CHUNK_W + off
            pk = ab_v[pl.ds(g, LANES)]
            ia = lax.bitwise_and(pk, 0x3FFF)
            ib = lax.shift_right_logical(pk, 14)
            p01 = c01_v[pl.ds(g, LANES)]
            p23 = c23_v[pl.ds(g, LANES)]
            c0 = lax.bitcast_convert_type(lax.shift_left(p01, 16), jnp.float32)
            c1 = lax.bitcast_convert_type(lax.bitwise_and(p01, _HI_MASK), jnp.float32)
            c2 = lax.bitcast_convert_type(lax.shift_left(p23, 16), jnp.float32)
            c3 = lax.bitcast_convert_type(lax.bitwise_and(p23, _HI_MASK), jnp.float32)
            va0 = plsc.load_gather(xb0, [ia])
            vb0 = plsc.load_gather(xb0, [ib])
            ob0[pl.ds(off, LANES)] = c0 + va0 * (c1 + c3 * vb0) + c2 * vb0
            va1 = plsc.load_gather(xb1, [ia])
            vb1 = plsc.load_gather(xb1, [ib])
            ob1[pl.ds(off, LANES)] = c0 + va1 * (c1 + c3 * vb1) + c2 * vb1

    def out_copies(pair, slot, ch):
        r0 = base + pair * 2
        col = ch * CHUNK_W
        return (
            pltpu.make_async_copy(obufs[slot][0],
                                  out_hbm.at[r0, pl.ds(col, CHUNK_W)],
                                  out_sems[slot]),
            pltpu.make_async_copy(obufs[slot][1],
                                  out_hbm.at[r0 + 1, pl.ds(col, CHUNK_W)],
                                  out_sems[slot]),
        )

    def pair_body(pair, buf):
        wait_in(pair, buf)
        if buf == 0:
            start_in(pair + 1, 1)
        else:
            @pl.when(pair + 1 < PAIRS_PER_WORKER)
            def _():
                start_in(pair + 1, 0)
        xb0, xb1 = xbufs[buf]
        for ch in range(NCHUNKS):
            slot = ch % 2
            @pl.when((pair * NCHUNKS + ch) >= 2)
            def _(slot=slot, ch=ch):
                for c in out_copies(pair, slot, ch):
                    c.wait()
            compute_chunk(xb0, xb1, slot, ch)
            for c in out_copies(pair, slot, ch):
                c.start()

    def pairs_fn(i, carry):
        pair_body(i * 2, 0)
        pair_body(i * 2 + 1, 1)
        return carry

    lax.fori_loop(0, PAIRS_PER_WORKER // 2, pairs_fn, 0)

    for c in out_copies(PAIRS_PER_WORKER - 1, 0, NCHUNKS - 2):
        c.wait()
    for c in out_copies(PAIRS_PER_WORKER - 1, 1, NCHUNKS - 1):
        c.wait()


def kernel(x, weights, a, b):
    c01, c23, ab = _coefficients(weights, a.astype(jnp.int32),
                                 b.astype(jnp.int32))
    return _sc_gather_combine(x, ab, c01, c23)


# speedup vs baseline: 1.3965x; 1.3965x over previous
import functools

import jax
import jax.numpy as jnp
from jax import lax
from jax.experimental import pallas as pl
from jax.experimental.pallas import tpu as pltpu
from jax.experimental.pallas import tpu_sc as plsc

IN_DIM = 16384
OUT_DIM = 16384
BATCH = 1024

NUM_CORES = 2
NUM_SUBCORES = 16
NUM_WORKERS = NUM_CORES * NUM_SUBCORES
ROWS_PER_WORKER = BATCH // NUM_WORKERS
PAIRS_PER_WORKER = ROWS_PER_WORKER // 2
LANES = 16
GROUPS = OUT_DIM // LANES
CHUNK_W = 2048
CHUNK_GROUPS = CHUNK_W // LANES
NCHUNKS = OUT_DIM // CHUNK_W
UNROLL = 4


def _coef_body(wt_ref, a_ref, b_ref, c01_ref, c23_ref, ab_ref):
    w = wt_ref[...]
    m = jnp.max(w, axis=0, keepdims=True)
    e = jnp.exp(w - m)
    p = e / jnp.sum(e, axis=0, keepdims=True)
    r = [p[i:i + 1, :] for i in range(16)]
    c0 = r[8] + r[9] + r[10] + r[11] + r[12] + r[13] + r[14] + r[15]
    c1 = (r[2] + r[3] + r[6] + r[7]) - (r[8] + r[9] + r[12] + r[13])
    c2 = (r[4] + r[5] + r[6] + r[7]) - (r[8] + r[9] + r[10] + r[11])
    c3 = (r[1] - r[2] - r[4] - 2.0 * r[6] - r[7]
          + r[8] + 2.0 * r[9] + r[11] + r[13] - r[14])

    def pack_pair(lo, hi):
        lo_b = lax.bitcast_convert_type(lo.astype(jnp.bfloat16), jnp.uint16)
        hi_b = lax.bitcast_convert_type(hi.astype(jnp.bfloat16), jnp.uint16)
        word = lax.bitwise_or(lo_b.astype(jnp.uint32),
                              lax.shift_left(hi_b.astype(jnp.uint32),
                                             jnp.uint32(16)))
        return lax.bitcast_convert_type(word, jnp.int32)

    c01_ref[...] = pack_pair(c0, c1)
    c23_ref[...] = pack_pair(c2, c3)
    ab_ref[...] = lax.bitwise_or(a_ref[...], lax.shift_left(b_ref[...], 14))


def _coefficients(weights, a, b):
    wt = weights.T
    c01, c23, ab = pl.pallas_call(
        _coef_body,
        out_shape=[
            jax.ShapeDtypeStruct((1, OUT_DIM), jnp.int32),
            jax.ShapeDtypeStruct((1, OUT_DIM), jnp.int32),
            jax.ShapeDtypeStruct((1, OUT_DIM), jnp.int32),
        ],
    )(wt, a.reshape(1, OUT_DIM), b.reshape(1, OUT_DIM))
    return c01.reshape(OUT_DIM), c23.reshape(OUT_DIM), ab.reshape(OUT_DIM)


_HI_MASK = -65536


@functools.partial(
    pl.kernel,
    mesh=plsc.VectorSubcoreMesh(core_axis_name="c", subcore_axis_name="s"),
    out_type=jax.ShapeDtypeStruct((BATCH, OUT_DIM), jnp.float32),
    compiler_params=pltpu.CompilerParams(needs_layout_passes=False),
    scratch_types=[
        pltpu.VMEM((OUT_DIM,), jnp.int32),
        pltpu.VMEM((OUT_DIM,), jnp.int32),
        pltpu.VMEM((OUT_DIM,), jnp.int32),
        pltpu.VMEM((IN_DIM,), jnp.float32),
        pltpu.VMEM((IN_DIM,), jnp.float32),
        pltpu.VMEM((IN_DIM,), jnp.float32),
        pltpu.VMEM((IN_DIM,), jnp.float32),
        pltpu.VMEM((CHUNK_W,), jnp.float32),
        pltpu.VMEM((CHUNK_W,), jnp.float32),
        pltpu.VMEM((CHUNK_W,), jnp.float32),
        pltpu.VMEM((CHUNK_W,), jnp.float32),
        pltpu.SemaphoreType.DMA,
        pltpu.SemaphoreType.DMA,
        pltpu.SemaphoreType.DMA,
        pltpu.SemaphoreType.DMA,
    ],
)
def _sc_gather_combine(x_hbm, ab_hbm, c01_hbm, c23_hbm, out_hbm,
                       ab_v, c01_v, c23_v,
                       x00, x01, x10, x11,
                       o00, o01, o10, o11,
                       in_sem0, in_sem1, out_sem0, out_sem1):
    wid = lax.axis_index("s") * NUM_CORES + lax.axis_index("c")
    base = wid * ROWS_PER_WORKER

    pltpu.sync_copy(ab_hbm, ab_v)
    pltpu.sync_copy(c01_hbm, c01_v)
    pltpu.sync_copy(c23_hbm, c23_v)

    xbufs = ((x00, x01), (x10, x11))
    obufs = ((o00, o10), (o01, o11))
    in_sems = (in_sem0, in_sem1)
    out_sems = (out_sem0, out_sem1)

    def in_copy(pair, buf):
        r0 = base + pair * 2
        return (
            pltpu.make_async_copy(x_hbm.at[r0], xbufs[buf][0], in_sems[buf]),
            pltpu.make_async_copy(x_hbm.at[r0 + 1], xbufs[buf][1], in_sems[buf]),
        )

    def start_in(pair, buf):
        for c in in_copy(pair, buf):
            c.start()

    def wait_in(pair, buf):
        for c in in_copy(pair, buf):
            c.wait()

    start_in(0, 0)

    def compute_chunk(xb0, xb1, slot, ch):
        ob0, ob1 = obufs[slot]

        @plsc.parallel_loop(0, CHUNK_GROUPS, unroll=UNROLL)
        def grp_fn(i):
            off = i * LANES
            g = ch * CHUNK_W + off
            pk = ab_v[pl.ds(g, LANES)]
            ia = lax.bitwise_and(pk, 0x3FFF)
            ib = lax.shift_right_logical(pk, 14)
            p01 = c01_v[pl.ds(g, LANES)]
            p23 = c23_v[pl.ds(g, LANES)]
            c0 = lax.bitcast_convert_type(lax.shift_left(p01, 16), jnp.float32)
            c1 = lax.bitcast_convert_type(lax.bitwise_and(p01, _HI_MASK), jnp.float32)
            c2 = lax.bitcast_convert_type(lax.shift_left(p23, 16), jnp.float32)
            c3 = lax.bitcast_convert_type(lax.bitwise_and(p23, _HI_MASK), jnp.float32)
            va0 = plsc.load_gather(xb0, [ia])
            vb0 = plsc.load_gather(xb0, [ib])
            ob0[pl.ds(off, LANES)] = c0 + va0 * (c1 + c3 * vb0) + c2 * vb0
            va1 = plsc.load_gather(xb1, [ia])
            vb1 = plsc.load_gather(xb1, [ib])
            ob1[pl.ds(off, LANES)] = c0 + va1 * (c1 + c3 * vb1) + c2 * vb1

    def out_copies(pair, slot, ch):
        r0 = base + pair * 2
        col = ch * CHUNK_W
        return (
            pltpu.make_async_copy(obufs[slot][0],
                                  out_hbm.at[r0, pl.ds(col, CHUNK_W)],
                                  out_sems[slot]),
            pltpu.make_async_copy(obufs[slot][1],
                                  out_hbm.at[r0 + 1, pl.ds(col, CHUNK_W)],
                                  out_sems[slot]),
        )

    def pair_body(pair, buf, first=False, last=False):
        wait_in(pair, buf)
        if not last:
            start_in(pair + 1, 1 - buf)
        xb0, xb1 = xbufs[buf]
        for ch in range(NCHUNKS):
            slot = ch % 2
            if not (first and ch < 2):
                for c in out_copies(pair, slot, ch):
                    c.wait()
            compute_chunk(xb0, xb1, slot, ch)
            for c in out_copies(pair, slot, ch):
                c.start()

    pair_body(0, 0, first=True)

    def pairs_fn(i, carry):
        pair = 1 + i * 2
        pair_body(pair, 1)
        pair_body(pair + 1, 0)
        return carry

    lax.fori_loop(0, (PAIRS_PER_WORKER - 2) // 2, pairs_fn, 0)
    pair_body(PAIRS_PER_WORKER - 1, 1, last=True)

    for c in out_copies(PAIRS_PER_WORKER - 1, 0, NCHUNKS - 2):
        c.wait()
    for c in out_copies(PAIRS_PER_WORKER - 1, 1, NCHUNKS - 1):
        c.wait()


def kernel(x, weights, a, b):
    c01, c23, ab = _coefficients(weights, a.astype(jnp.int32),
                                 b.astype(jnp.int32))
    return _sc_gather_combine(x, ab, c01, c23)


# speedup vs baseline: 1.4888x; 1.0661x over previous
import functools

import jax
import jax.numpy as jnp
from jax import lax
from jax.experimental import pallas as pl
from jax.experimental.pallas import tpu as pltpu
from jax.experimental.pallas import tpu_sc as plsc

IN_DIM = 16384
OUT_DIM = 16384
BATCH = 1024

NUM_CORES = 2
NUM_SUBCORES = 16
NUM_WORKERS = NUM_CORES * NUM_SUBCORES
ROWS_PER_WORKER = BATCH // NUM_WORKERS
PAIRS_PER_WORKER = ROWS_PER_WORKER // 2
LANES = 16
GROUPS = OUT_DIM // LANES
CHUNK_W = 2048
CHUNK_GROUPS = CHUNK_W // LANES
NCHUNKS = OUT_DIM // CHUNK_W
UNROLL = 4


def _coef_body(wt_ref, a_ref, b_ref, c01_ref, c23_ref, ab_ref):
    w = wt_ref[...]
    m = jnp.max(w, axis=0, keepdims=True)
    e = jnp.exp(w - m)
    p = e / jnp.sum(e, axis=0, keepdims=True)
    r = [p[i:i + 1, :] for i in range(16)]
    c0 = r[8] + r[9] + r[10] + r[11] + r[12] + r[13] + r[14] + r[15]
    c1 = (r[2] + r[3] + r[6] + r[7]) - (r[8] + r[9] + r[12] + r[13])
    c2 = (r[4] + r[5] + r[6] + r[7]) - (r[8] + r[9] + r[10] + r[11])
    c3 = (r[1] - r[2] - r[4] - 2.0 * r[6] - r[7]
          + r[8] + 2.0 * r[9] + r[11] + r[13] - r[14])

    def pack_pair(lo, hi):
        lo_b = lax.bitcast_convert_type(lo.astype(jnp.bfloat16), jnp.uint16)
        hi_b = lax.bitcast_convert_type(hi.astype(jnp.bfloat16), jnp.uint16)
        word = lax.bitwise_or(lo_b.astype(jnp.uint32),
                              lax.shift_left(hi_b.astype(jnp.uint32),
                                             jnp.uint32(16)))
        return lax.bitcast_convert_type(word, jnp.int32)

    c01_ref[...] = pack_pair(c0, c1)
    c23_ref[...] = pack_pair(c2, c3)
    ab_ref[...] = lax.bitwise_or(a_ref[...], lax.shift_left(b_ref[...], 14))


def _coefficients(weights, a, b):
    wt = weights.T
    c01, c23, ab = pl.pallas_call(
        _coef_body,
        out_shape=[
            jax.ShapeDtypeStruct((1, OUT_DIM), jnp.int32),
            jax.ShapeDtypeStruct((1, OUT_DIM), jnp.int32),
            jax.ShapeDtypeStruct((1, OUT_DIM), jnp.int32),
        ],
    )(wt, a.reshape(1, OUT_DIM), b.reshape(1, OUT_DIM))
    return c01.reshape(OUT_DIM), c23.reshape(OUT_DIM), ab.reshape(OUT_DIM)


_HI_MASK = -65536


@functools.partial(
    pl.kernel,
    mesh=plsc.VectorSubcoreMesh(core_axis_name="c", subcore_axis_name="s"),
    out_type=jax.ShapeDtypeStruct((BATCH, OUT_DIM), jnp.float32),
    compiler_params=pltpu.CompilerParams(needs_layout_passes=False),
    scratch_types=[
        pltpu.VMEM((OUT_DIM,), jnp.int32),
        pltpu.VMEM((OUT_DIM,), jnp.int32),
        pltpu.VMEM((OUT_DIM,), jnp.int32),
        pltpu.VMEM((IN_DIM,), jnp.float32),
        pltpu.VMEM((IN_DIM,), jnp.float32),
        pltpu.VMEM((IN_DIM,), jnp.float32),
        pltpu.VMEM((IN_DIM,), jnp.float32),
        pltpu.VMEM((CHUNK_W,), jnp.float32),
        pltpu.VMEM((CHUNK_W,), jnp.float32),
        pltpu.VMEM((CHUNK_W,), jnp.float32),
        pltpu.VMEM((CHUNK_W,), jnp.float32),
        pltpu.SemaphoreType.DMA,
        pltpu.SemaphoreType.DMA,
        pltpu.SemaphoreType.DMA,
        pltpu.SemaphoreType.DMA,
    ],
)
def _sc_gather_combine(x_hbm, ab_hbm, c01_hbm, c23_hbm, out_hbm,
                       ab_v, c01_v, c23_v,
                       x00, x01, x10, x11,
                       o00, o01, o10, o11,
                       in_sem0, in_sem1, out_sem0, out_sem1):
    wid = lax.axis_index("s") * NUM_CORES + lax.axis_index("c")
    base = wid * ROWS_PER_WORKER

    pltpu.sync_copy(ab_hbm, ab_v)
    pltpu.sync_copy(c01_hbm, c01_v)
    pltpu.sync_copy(c23_hbm, c23_v)

    xbufs = ((x00, x01), (x10, x11))
    obufs = ((o00, o10), (o01, o11))
    in_sems = (in_sem0, in_sem1)
    out_sems = (out_sem0, out_sem1)

    def in_copy(pair, buf):
        r0 = base + pair * 2
        return (
            pltpu.make_async_copy(x_hbm.at[r0], xbufs[buf][0], in_sems[buf]),
            pltpu.make_async_copy(x_hbm.at[r0 + 1], xbufs[buf][1], in_sems[buf]),
        )

    def start_in(pair, buf):
        for c in in_copy(pair, buf):
            c.start()

    def wait_in(pair, buf):
        for c in in_copy(pair, buf):
            c.wait()

    start_in(0, 0)

    def compute_chunk(xb0, xb1, slot, ch):
        ob0, ob1 = obufs[slot]

        @plsc.parallel_loop(0, CHUNK_GROUPS, unroll=UNROLL)
        def grp_fn(i):
            off = i * LANES
            g = ch * CHUNK_W + off
            pk = ab_v[pl.ds(g, LANES)]
            ia = lax.bitwise_and(pk, 0x3FFF)
            ib = lax.shift_right_logical(pk, 14)
            p01 = c01_v[pl.ds(g, LANES)]
            p23 = c23_v[pl.ds(g, LANES)]
            c0 = lax.bitcast_convert_type(lax.shift_left(p01, 16), jnp.float32)
            c1 = lax.bitcast_convert_type(lax.bitwise_and(p01, _HI_MASK), jnp.float32)
            c2 = lax.bitcast_convert_type(lax.shift_left(p23, 16), jnp.float32)
            c3 = lax.bitcast_convert_type(lax.bitwise_and(p23, _HI_MASK), jnp.float32)
            va0 = plsc.load_gather(xb0, [ia])
            vb0 = plsc.load_gather(xb0, [ib])
            ob0[pl.ds(off, LANES)] = c0 + va0 * (c1 + c3 * vb0) + c2 * vb0
            va1 = plsc.load_gather(xb1, [ia])
            vb1 = plsc.load_gather(xb1, [ib])
            ob1[pl.ds(off, LANES)] = c0 + va1 * (c1 + c3 * vb1) + c2 * vb1

    def out_copies(pair, slot, ch):
        r0 = base + pair * 2
        col = ch * CHUNK_W
        return (
            pltpu.make_async_copy(obufs[slot][0],
                                  out_hbm.at[r0, pl.ds(col, CHUNK_W)],
                                  out_sems[slot]),
            pltpu.make_async_copy(obufs[slot][1],
                                  out_hbm.at[r0 + 1, pl.ds(col, CHUNK_W)],
                                  out_sems[slot]),
        )

    def pair_body(pair, buf):
        wait_in(pair, buf)
        if buf == 0:
            start_in(pair + 1, 1)
        else:
            @pl.when(pair + 1 < PAIRS_PER_WORKER)
            def _():
                start_in(pair + 1, 0)
        xb0, xb1 = xbufs[buf]

        def chunk_phase(ch, slot):
            @pl.when((pair * NCHUNKS + ch) >= 2)
            def _():
                for c in out_copies(pair, slot, ch):
                    c.wait()
            compute_chunk(xb0, xb1, slot, ch)
            for c in out_copies(pair, slot, ch):
                c.start()

        def chunks_fn(k, carry):
            chunk_phase(k * 2, 0)
            chunk_phase(k * 2 + 1, 1)
            return carry

        lax.fori_loop(0, NCHUNKS // 2, chunks_fn, 0)

    def pairs_fn(i, carry):
        pair_body(i * 2, 0)
        pair_body(i * 2 + 1, 1)
        return carry

    lax.fori_loop(0, PAIRS_PER_WORKER // 2, pairs_fn, 0)

    for c in out_copies(PAIRS_PER_WORKER - 1, 0, NCHUNKS - 2):
        c.wait()
    for c in out_copies(PAIRS_PER_WORKER - 1, 1, NCHUNKS - 1):
        c.wait()


def kernel(x, weights, a, b):
    c01, c23, ab = _coefficients(weights, a.astype(jnp.int32),
                                 b.astype(jnp.int32))
    return _sc_gather_combine(x, ab, c01, c23)


# speedup vs baseline: 1.5139x; 1.0168x over previous
import functools

import jax
import jax.numpy as jnp
from jax import lax
from jax.experimental import pallas as pl
from jax.experimental.pallas import tpu as pltpu
from jax.experimental.pallas import tpu_sc as plsc

IN_DIM = 16384
OUT_DIM = 16384
BATCH = 1024

NUM_CORES = 2
NUM_SUBCORES = 16
NUM_WORKERS = NUM_CORES * NUM_SUBCORES
ROWS_PER_WORKER = BATCH // NUM_WORKERS
PAIRS_PER_WORKER = ROWS_PER_WORKER // 2
LANES = 16
GROUPS = OUT_DIM // LANES
CHUNK_W = 2048
CHUNK_GROUPS = CHUNK_W // LANES
NCHUNKS = OUT_DIM // CHUNK_W
UNROLL = 4


def _coef_body(wt_ref, a_ref, b_ref, c01_ref, c23_ref, ab_ref):
    w = wt_ref[...]
    m = jnp.max(w, axis=0, keepdims=True)
    e = jnp.exp(w - m)
    p = e / jnp.sum(e, axis=0, keepdims=True)
    r = [p[i:i + 1, :] for i in range(16)]
    c0 = r[8] + r[9] + r[10] + r[11] + r[12] + r[13] + r[14] + r[15]
    c1 = (r[2] + r[3] + r[6] + r[7]) - (r[8] + r[9] + r[12] + r[13])
    c2 = (r[4] + r[5] + r[6] + r[7]) - (r[8] + r[9] + r[10] + r[11])
    c3 = (r[1] - r[2] - r[4] - 2.0 * r[6] - r[7]
          + r[8] + 2.0 * r[9] + r[11] + r[13] - r[14])

    def pack_pair(lo, hi):
        lo_b = lax.bitcast_convert_type(lo.astype(jnp.bfloat16), jnp.uint16)
        hi_b = lax.bitcast_convert_type(hi.astype(jnp.bfloat16), jnp.uint16)
        word = lax.bitwise_or(lo_b.astype(jnp.uint32),
                              lax.shift_left(hi_b.astype(jnp.uint32),
                                             jnp.uint32(16)))
        return lax.bitcast_convert_type(word, jnp.int32)

    c01_ref[...] = pack_pair(c0, c1)
    c23_ref[...] = pack_pair(c2, c3)
    ab_ref[...] = lax.bitwise_or(a_ref[...], lax.shift_left(b_ref[...], 14))


def _coefficients(weights, a, b):
    wt = weights.T
    c01, c23, ab = pl.pallas_call(
        _coef_body,
        out_shape=[
            jax.ShapeDtypeStruct((1, OUT_DIM), jnp.int32),
            jax.ShapeDtypeStruct((1, OUT_DIM), jnp.int32),
            jax.ShapeDtypeStruct((1, OUT_DIM), jnp.int32),
        ],
    )(wt, a.reshape(1, OUT_DIM), b.reshape(1, OUT_DIM))
    return c01.reshape(OUT_DIM), c23.reshape(OUT_DIM), ab.reshape(OUT_DIM)


_HI_MASK = -65536


@functools.partial(
    pl.kernel,
    mesh=plsc.VectorSubcoreMesh(core_axis_name="c", subcore_axis_name="s"),
    out_type=jax.ShapeDtypeStruct((BATCH, OUT_DIM), jnp.float32),
    compiler_params=pltpu.CompilerParams(needs_layout_passes=False),
    scratch_types=[
        pltpu.VMEM((OUT_DIM,), jnp.int32),
        pltpu.VMEM((OUT_DIM,), jnp.int32),
        pltpu.VMEM((OUT_DIM,), jnp.int32),
        pltpu.VMEM((IN_DIM,), jnp.float32),
        pltpu.VMEM((IN_DIM,), jnp.float32),
        pltpu.VMEM((IN_DIM,), jnp.float32),
        pltpu.VMEM((IN_DIM,), jnp.float32),
        pltpu.VMEM((CHUNK_W,), jnp.float32),
        pltpu.VMEM((CHUNK_W,), jnp.float32),
        pltpu.VMEM((CHUNK_W,), jnp.float32),
        pltpu.VMEM((CHUNK_W,), jnp.float32),
        pltpu.SemaphoreType.DMA,
        pltpu.SemaphoreType.DMA,
        pltpu.SemaphoreType.DMA,
        pltpu.SemaphoreType.DMA,
    ],
)
def _sc_gather_combine(x_hbm, ab_hbm, c01_hbm, c23_hbm, out_hbm,
                       ab_v, c01_v, c23_v,
                       x00, x01, x10, x11,
                       o00, o01, o10, o11,
                       in_sem0, in_sem1, out_sem0, out_sem1):
    wid = lax.axis_index("s") * NUM_CORES + lax.axis_index("c")
    base = wid * ROWS_PER_WORKER

    xbufs = ((x00, x01), (x10, x11))
    obufs = ((o00, o10), (o01, o11))
    in_sems = (in_sem0, in_sem1)
    out_sems = (out_sem0, out_sem1)

    def in_copy(pair, buf):
        r0 = base + pair * 2
        return (
            pltpu.make_async_copy(x_hbm.at[r0], xbufs[buf][0], in_sems[buf]),
            pltpu.make_async_copy(x_hbm.at[r0 + 1], xbufs[buf][1], in_sems[buf]),
        )

    def start_in(pair, buf):
        for c in in_copy(pair, buf):
            c.start()

    def wait_in(pair, buf):
        for c in in_copy(pair, buf):
            c.wait()

    const_copies = (
        pltpu.make_async_copy(ab_hbm, ab_v, out_sem0),
        pltpu.make_async_copy(c01_hbm, c01_v, out_sem0),
        pltpu.make_async_copy(c23_hbm, c23_v, out_sem0),
    )
    for c in const_copies:
        c.start()
    start_in(0, 0)
    for c in const_copies:
        c.wait()

    def compute_chunk(xb0, xb1, slot, ch):
        ob0, ob1 = obufs[slot]

        @plsc.parallel_loop(0, CHUNK_GROUPS, unroll=UNROLL)
        def grp_fn(i):
            off = i * LANES
            g = ch * CHUNK_W + off
            pk = ab_v[pl.ds(g, LANES)]
            ia = lax.bitwise_and(pk, 0x3FFF)
            ib = lax.shift_right_logical(pk, 14)
            p01 = c01_v[pl.ds(g, LANES)]
            p23 = c23_v[pl.ds(g, LANES)]
            c0 = lax.bitcast_convert_type(lax.shift_left(p01, 16), jnp.float32)
            c1 = lax.bitcast_convert_type(lax.bitwise_and(p01, _HI_MASK), jnp.float32)
            c2 = lax.bitcast_convert_type(lax.shift_left(p23, 16), jnp.float32)
            c3 = lax.bitcast_convert_type(lax.bitwise_and(p23, _HI_MASK), jnp.float32)
            va0 = plsc.load_gather(xb0, [ia])
            vb0 = plsc.load_gather(xb0, [ib])
            ob0[pl.ds(off, LANES)] = c0 + va0 * (c1 + c3 * vb0) + c2 * vb0
            va1 = plsc.load_gather(xb1, [ia])
            vb1 = plsc.load_gather(xb1, [ib])
            ob1[pl.ds(off, LANES)] = c0 + va1 * (c1 + c3 * vb1) + c2 * vb1

    def out_copies(pair, slot, ch):
        r0 = base + pair * 2
        col = ch * CHUNK_W
        return (
            pltpu.make_async_copy(obufs[slot][0],
                                  out_hbm.at[r0, pl.ds(col, CHUNK_W)],
                                  out_sems[slot]),
            pltpu.make_async_copy(obufs[slot][1],
                                  out_hbm.at[r0 + 1, pl.ds(col, CHUNK_W)],
                                  out_sems[slot]),
        )

    def pair_body(pair, buf):
        wait_in(pair, buf)
        if buf == 0:
            start_in(pair + 1, 1)
        else:
            @pl.when(pair + 1 < PAIRS_PER_WORKER)
            def _():
                start_in(pair + 1, 0)
        xb0, xb1 = xbufs[buf]

        def chunk_phase(ch, slot):
            @pl.when((pair * NCHUNKS + ch) >= 2)
            def _():
                for c in out_copies(pair, slot, ch):
                    c.wait()
            compute_chunk(xb0, xb1, slot, ch)
            for c in out_copies(pair, slot, ch):
                c.start()

        def chunks_fn(k, carry):
            chunk_phase(k * 2, 0)
            chunk_phase(k * 2 + 1, 1)
            return carry

        lax.fori_loop(0, NCHUNKS // 2, chunks_fn, 0)

    def pairs_fn(i, carry):
        pair_body(i * 2, 0)
        pair_body(i * 2 + 1, 1)
        return carry

    lax.fori_loop(0, PAIRS_PER_WORKER // 2, pairs_fn, 0)

    for c in out_copies(PAIRS_PER_WORKER - 1, 0, NCHUNKS - 2):
        c.wait()
    for c in out_copies(PAIRS_PER_WORKER - 1, 1, NCHUNKS - 1):
        c.wait()


def kernel(x, weights, a, b):
    c01, c23, ab = _coefficients(weights, a.astype(jnp.int32),
                                 b.astype(jnp.int32))
    return _sc_gather_combine(x, ab, c01, c23)
